# baseline (speedup 1.0000x reference)
import functools

import jax
import jax.numpy as jnp
from jax.experimental import pallas as pl
from jax.experimental.pallas import tpu as pltpu

B = 4096
INPUT_DIM = 1024
HIDDEN_DIM = 2048
K = 8192
D = 32
L = 16

ENC_BT = 512
VQ_BT = 512
DEC_BT = 512


def _encoder_kernel(x_ref, w1_ref, b1_ref, w2_ref, b2_ref, z_ref):
    h = jnp.maximum(
        jnp.dot(x_ref[...], w1_ref[...], preferred_element_type=jnp.float32)
        + b1_ref[...],
        0.0,
    )
    z_ref[...] = (
        jnp.dot(h, w2_ref[...], preferred_element_type=jnp.float32) + b2_ref[...]
    )


def _encvq_kernel(
    x_ref, w1_ref, b1_ref, w2_ref, b2_ref, csq_ref, cbt_ref, z_ref, idx_ref
):
    h = jnp.maximum(
        jnp.dot(x_ref[...], w1_ref[...], preferred_element_type=jnp.float32)
        + b1_ref[...],
        0.0,
    )
    z_ref[...] = (
        jnp.dot(h, w2_ref[...], preferred_element_type=jnp.float32) + b2_ref[...]
    )
    iota_k = jax.lax.broadcasted_iota(jnp.int32, (1, K), 1)
    for l in range(L):
        zi = z_ref[:, l * D : (l + 1) * D]
        zsq_l = jnp.sum(zi * zi, axis=1, keepdims=True)
        dot2 = jnp.dot(
            (zi + zi).astype(jnp.bfloat16),
            cbt_ref[...],
            preferred_element_type=jnp.float32,
        )
        dist = (zsq_l + csq_ref[...]) - dot2
        minval = jnp.min(dist, axis=1, keepdims=True)
        idx = jnp.min(jnp.where(dist == minval, iota_k, K), axis=1).astype(jnp.int32)
        idx_ref[:, l] = idx


def _decoder_kernel(zq_ref, w3_ref, b3_ref, w4_ref, b4_ref, out_ref):
    h = jnp.maximum(
        jnp.dot(
            zq_ref[...].astype(jnp.bfloat16),
            w3_ref[...],
            preferred_element_type=jnp.float32,
        )
        + b3_ref[...],
        0.0,
    )
    out_ref[...] = (
        jnp.dot(
            h.astype(jnp.bfloat16), w4_ref[...], preferred_element_type=jnp.float32
        )
        + b4_ref[...]
    )


SC_DP = 128
SC_CHUNK = 512


def _sc_gather(table, idx_flat):
    from jax.experimental.pallas import tpu_sc as plsc

    n = idx_flat.shape[0]
    info = plsc.get_sparse_core_info()
    nw = info.num_cores * info.num_subcores
    b_per_w = n // nw
    nchunk = b_per_w // SC_CHUNK
    mesh = plsc.VectorSubcoreMesh(core_axis_name="c", subcore_axis_name="s")
    table_pad = jnp.pad(table, ((0, 0), (0, SC_DP - D)))

    @functools.partial(
        pl.kernel,
        mesh=mesh,
        out_type=jax.ShapeDtypeStruct((n, SC_DP), jnp.float32),
        scratch_types=[
            pltpu.VMEM((SC_CHUNK,), jnp.int32),
            pltpu.VMEM((SC_CHUNK, SC_DP), jnp.float32),
            pltpu.SemaphoreType.DMA,
        ],
    )
    def gk(table_hbm, idx_hbm, out_hbm, idx_v, rows_v, sem):
        wid = jax.lax.axis_index("s") * info.num_cores + jax.lax.axis_index("c")
        base = wid * b_per_w
        for c in range(nchunk):
            off = base + c * SC_CHUNK
            pltpu.sync_copy(idx_hbm.at[pl.ds(off, SC_CHUNK)], idx_v)
            pltpu.async_copy(table_hbm.at[idx_v], rows_v, sem).wait()
            pltpu.sync_copy(rows_v, out_hbm.at[pl.ds(off, SC_CHUNK)])

    return gk(table_pad, idx_flat)[:, :D]


def _full(shape):
    return pl.BlockSpec(shape, lambda i: (0,) * len(shape))


def _rows(bt, cols):
    return pl.BlockSpec((bt, cols), lambda i: (i, 0))


@jax.jit
def kernel(x, W1, b1, W2, b2, codebook, W3, b3, W4, b4):
    b1r = b1.reshape(1, HIDDEN_DIM)
    b2r = b2.reshape(1, L * D)
    b3r = b3.reshape(1, HIDDEN_DIM)
    b4r = b4.reshape(1, INPUT_DIM)

    c_sq = (codebook**2).sum(axis=1)
    cbt_bf16 = codebook.T.astype(jnp.bfloat16)

    z, indices = pl.pallas_call(
        _encvq_kernel,
        grid=(B // VQ_BT,),
        in_specs=[
            _rows(VQ_BT, INPUT_DIM),
            _full((INPUT_DIM, HIDDEN_DIM)),
            _full((1, HIDDEN_DIM)),
            _full((HIDDEN_DIM, L * D)),
            _full((1, L * D)),
            _full((1, K)),
            _full((D, K)),
        ],
        out_specs=[
            _rows(VQ_BT, L * D),
            _rows(VQ_BT, L),
        ],
        out_shape=[
            jax.ShapeDtypeStruct((B, L * D), jnp.float32),
            jax.ShapeDtypeStruct((B, L), jnp.int32),
        ],
        compiler_params=pltpu.CompilerParams(
            dimension_semantics=("parallel",),
        ),
    )(x, W1, b1r, W2, b2r, c_sq.reshape(1, K), cbt_bf16)
    z3 = z.reshape(B, L, D)

    z_q = _sc_gather(codebook, indices.reshape(B * L)).reshape(B, L * D)

    x_recon = pl.pallas_call(
        _decoder_kernel,
        grid=(B // DEC_BT,),
        in_specs=[
            _rows(DEC_BT, L * D),
            _full((L * D, HIDDEN_DIM)),
            _full((1, HIDDEN_DIM)),
            _full((HIDDEN_DIM, INPUT_DIM)),
            _full((1, INPUT_DIM)),
        ],
        out_specs=_rows(DEC_BT, INPUT_DIM),
        out_shape=jax.ShapeDtypeStruct((B, INPUT_DIM), jnp.float32),
        compiler_params=pltpu.CompilerParams(
            dimension_semantics=("parallel",),
        ),
    )(z_q, W3.astype(jnp.bfloat16), b3r, W4.astype(jnp.bfloat16), b4r)

    return (x_recon, z3, z_q, indices)


# speedup vs baseline: 1.0513x; 1.0513x over previous
import functools

import jax
import jax.numpy as jnp
from jax.experimental import pallas as pl
from jax.experimental.pallas import tpu as pltpu

B = 4096
INPUT_DIM = 1024
HIDDEN_DIM = 2048
K = 8192
D = 32
L = 16

ENC_BT = 512
VQ_BT = 512
DEC_BT = 512


def _encoder_kernel(x_ref, w1_ref, b1_ref, w2_ref, b2_ref, z_ref):
    h = jnp.maximum(
        jnp.dot(x_ref[...], w1_ref[...], preferred_element_type=jnp.float32)
        + b1_ref[...],
        0.0,
    )
    z_ref[...] = (
        jnp.dot(h, w2_ref[...], preferred_element_type=jnp.float32) + b2_ref[...]
    )


def _vq_kernel(z_ref, csq_ref, cbt_ref, idx_ref):
    iota_k = jax.lax.broadcasted_iota(jnp.int32, (1, K), 1)
    for l in range(L):
        zi = z_ref[:, l * D : (l + 1) * D]
        zsq_l = jnp.sum(zi * zi, axis=1, keepdims=True)
        dot2 = jnp.dot(
            (zi + zi).astype(jnp.bfloat16),
            cbt_ref[...],
            preferred_element_type=jnp.float32,
        )
        dist = (zsq_l + csq_ref[...]) - dot2
        minval = jnp.min(dist, axis=1, keepdims=True)
        idx = jnp.min(jnp.where(dist == minval, iota_k, K), axis=1).astype(jnp.int32)
        idx_ref[:, l] = idx


def _decoder_kernel(zq_ref, w3_ref, b3_ref, w4_ref, b4_ref, out_ref):
    h = jnp.maximum(
        jnp.dot(
            zq_ref[...].astype(jnp.bfloat16),
            w3_ref[...],
            preferred_element_type=jnp.float32,
        )
        + b3_ref[...],
        0.0,
    )
    out_ref[...] = (
        jnp.dot(
            h.astype(jnp.bfloat16), w4_ref[...], preferred_element_type=jnp.float32
        )
        + b4_ref[...]
    )


SC_DP = 128
SC_CHUNK = 512


def _sc_gather(table, idx_flat):
    from jax.experimental.pallas import tpu_sc as plsc

    n = idx_flat.shape[0]
    info = plsc.get_sparse_core_info()
    nw = info.num_cores * info.num_subcores
    b_per_w = n // nw
    nchunk = b_per_w // SC_CHUNK
    mesh = plsc.VectorSubcoreMesh(core_axis_name="c", subcore_axis_name="s")
    table_pad = jnp.pad(table, ((0, 0), (0, SC_DP - D)))

    @functools.partial(
        pl.kernel,
        mesh=mesh,
        out_type=jax.ShapeDtypeStruct((n, SC_DP), jnp.float32),
        scratch_types=[
            pltpu.VMEM((SC_CHUNK,), jnp.int32),
            pltpu.VMEM((SC_CHUNK, SC_DP), jnp.float32),
            pltpu.SemaphoreType.DMA,
        ],
    )
    def gk(table_hbm, idx_hbm, out_hbm, idx_v, rows_v, sem):
        wid = jax.lax.axis_index("s") * info.num_cores + jax.lax.axis_index("c")
        base = wid * b_per_w
        for c in range(nchunk):
            off = base + c * SC_CHUNK
            pltpu.sync_copy(idx_hbm.at[pl.ds(off, SC_CHUNK)], idx_v)
            pltpu.async_copy(table_hbm.at[idx_v], rows_v, sem).wait()
            pltpu.sync_copy(rows_v, out_hbm.at[pl.ds(off, SC_CHUNK)])

    return gk(table_pad, idx_flat)[:, :D]


def _full(shape):
    return pl.BlockSpec(shape, lambda i: (0,) * len(shape))


def _rows(bt, cols):
    return pl.BlockSpec((bt, cols), lambda i: (i, 0))


@jax.jit
def kernel(x, W1, b1, W2, b2, codebook, W3, b3, W4, b4):
    b1r = b1.reshape(1, HIDDEN_DIM)
    b2r = b2.reshape(1, L * D)
    b3r = b3.reshape(1, HIDDEN_DIM)
    b4r = b4.reshape(1, INPUT_DIM)

    c_sq = (codebook**2).sum(axis=1)
    cbt_bf16 = codebook.T.astype(jnp.bfloat16)

    z = pl.pallas_call(
        _encoder_kernel,
        grid=(B // ENC_BT,),
        in_specs=[
            _rows(ENC_BT, INPUT_DIM),
            _full((INPUT_DIM, HIDDEN_DIM)),
            _full((1, HIDDEN_DIM)),
            _full((HIDDEN_DIM, L * D)),
            _full((1, L * D)),
        ],
        out_specs=_rows(ENC_BT, L * D),
        out_shape=jax.ShapeDtypeStruct((B, L * D), jnp.float32),
        compiler_params=pltpu.CompilerParams(
            dimension_semantics=("parallel",),
        ),
    )(x, W1, b1r, W2, b2r)
    z3 = z.reshape(B, L, D)

    indices = pl.pallas_call(
        _vq_kernel,
        grid=(B // VQ_BT,),
        in_specs=[
            _rows(VQ_BT, L * D),
            _full((1, K)),
            _full((D, K)),
        ],
        out_specs=_rows(VQ_BT, L),
        out_shape=jax.ShapeDtypeStruct((B, L), jnp.int32),
        compiler_params=pltpu.CompilerParams(
            dimension_semantics=("parallel",),
        ),
    )(z, c_sq.reshape(1, K), cbt_bf16)

    z_q = _sc_gather(codebook, indices.reshape(B * L)).reshape(B, L * D)

    x_recon = pl.pallas_call(
        _decoder_kernel,
        grid=(B // DEC_BT,),
        in_specs=[
            _rows(DEC_BT, L * D),
            _full((L * D, HIDDEN_DIM)),
            _full((1, HIDDEN_DIM)),
            _full((HIDDEN_DIM, INPUT_DIM)),
            _full((1, INPUT_DIM)),
        ],
        out_specs=_rows(DEC_BT, INPUT_DIM),
        out_shape=jax.ShapeDtypeStruct((B, INPUT_DIM), jnp.float32),
        compiler_params=pltpu.CompilerParams(
            dimension_semantics=("parallel",),
        ),
    )(z_q, W3.astype(jnp.bfloat16), b3r, W4.astype(jnp.bfloat16), b4r)

    return (x_recon, z3, z_q, indices)
